# baseline (speedup 1.0000x reference)
import functools

import jax
import jax.numpy as jnp
from jax import lax
from jax.experimental import pallas as pl
from jax.experimental.pallas import tpu as pltpu
from jax.experimental.pallas import tpu_sc as plsc


def _sc_partials(feature, label, *, n, num_workers=32):
    rows = n // 64
    base = rows // num_workers
    rem = rows % num_workers
    cg = 1
    for d in range(650, 0, -1):
        if base % d == 0:
            cg = d
            break
    k_chunks = base // cg

    mesh = plsc.VectorSubcoreMesh(
        core_axis_name="c", subcore_axis_name="s",
        num_cores=2, num_subcores=num_workers // 2)

    @functools.partial(
        pl.kernel,
        out_type=jax.ShapeDtypeStruct((num_workers, 4, 16), jnp.float32),
        mesh=mesh,
        compiler_params=pltpu.CompilerParams(needs_layout_passes=False),
        scratch_types=[
            pltpu.VMEM((cg * 64,), jnp.float32),
            pltpu.VMEM((cg * 64,), jnp.float32),
            pltpu.VMEM((cg * 64,), jnp.float32),
            pltpu.VMEM((32,), jnp.float32),
            pltpu.VMEM((16,), jnp.float32),
            pltpu.VMEM((16,), jnp.float32),
            pltpu.VMEM((32,), jnp.float32),
            pltpu.VMEM((16,), jnp.float32),
            pltpu.VMEM((16,), jnp.float32),
            pltpu.VMEM((4, 16), jnp.float32),
        ],
    )
    def sc_kernel(fx_hbm, fy_hbm, label_hbm, part_hbm, fxbuf, fybuf, lbuf,
                  s1, s2, cnt, s1b, s2b, cntb, obuf):
        wid = lax.axis_index("s") * 2 + lax.axis_index("c")
        rstart = wid * base + jnp.minimum(wid, rem)

        iota = lax.iota(jnp.int32, 16)
        dbl = iota + iota
        zeros = jnp.zeros((16,), jnp.float32)
        ones = jnp.ones((16,), jnp.float32)

        s1[pl.ds(0, 16)] = zeros
        s1[pl.ds(16, 16)] = zeros
        s2[...] = zeros
        cnt[...] = zeros
        s1b[pl.ds(0, 16)] = zeros
        s1b[pl.ds(16, 16)] = zeros
        s2b[...] = zeros
        cntb[...] = zeros

        def group_into(g, t1, t2, tc):
            b16 = g * 16
            lab = lbuf[pl.ds(b16, 16)]
            labi = lab.astype(jnp.int32)
            vx = fxbuf[pl.ds(b16, 16)]
            vy = fybuf[pl.ds(b16, 16)]
            b2 = labi + labi
            plsc.addupdate_scatter(tc, [labi], ones)
            plsc.addupdate_scatter(t1, [b2], vx)
            plsc.addupdate_scatter(t1, [b2 + 1], vy)
            plsc.addupdate_scatter(t2, [labi], vx * vx + vy * vy)

        def group_body(g, _):
            group_into(g, s1, s2, cnt)
            return _

        def pair_body(p, _):
            g0 = p + p
            group_into(g0, s1, s2, cnt)
            group_into(g0 + 1, s1b, s2b, cntb)
            return _

        def chunk_body(k, _):
            soff = (rstart + k * cg) * 64
            pltpu.sync_copy(fx_hbm.at[pl.ds(soff, cg * 64)], fxbuf)
            pltpu.sync_copy(fy_hbm.at[pl.ds(soff, cg * 64)], fybuf)
            pltpu.sync_copy(label_hbm.at[pl.ds(soff, cg * 64)], lbuf)
            lax.fori_loop(0, cg * 2, pair_body, None, unroll=2)
            return _

        lax.fori_loop(0, k_chunks, chunk_body, None)

        @pl.when(wid < rem)
        def _tail():
            soff = (rstart + base) * 64
            pltpu.sync_copy(fx_hbm.at[pl.ds(soff, 64)],
                            fxbuf.at[pl.ds(0, 64)])
            pltpu.sync_copy(fy_hbm.at[pl.ds(soff, 64)],
                            fybuf.at[pl.ds(0, 64)])
            pltpu.sync_copy(label_hbm.at[pl.ds(soff, 64)],
                            lbuf.at[pl.ds(0, 64)])
            lax.fori_loop(0, 4, group_body, None, unroll=2)

        s1[pl.ds(0, 16)] += s1b[pl.ds(0, 16)]
        s1[pl.ds(16, 16)] += s1b[pl.ds(16, 16)]
        s2[...] += s2b[...]
        cnt[...] += cntb[...]
        obuf[0, :] = plsc.load_gather(s1, [dbl])
        obuf[1, :] = plsc.load_gather(s1, [dbl + 1])
        obuf[2, :] = s2[...]
        obuf[3, :] = cnt[...]
        pltpu.sync_copy(obuf, part_hbm.at[wid])

    return sc_kernel(feature[:, 0], feature[:, 1], label)


def _tc_combine(partials, center_t, lam, *, n):
    def body(p_ref, ct_ref, lam_ref, o_ref):
        r = jnp.sum(p_ref[...], axis=0)
        s1x = r[0:1, :]
        s1y = r[1:2, :]
        s2c = r[2:3, :]
        cntc = r[3:4, :]
        cx = ct_ref[0:1, :]
        cy = ct_ref[1:2, :]
        num = s2c - 2.0 * (cx * s1x + cy * s1y)
        per = jnp.where(cntc > 0.0,
                        num / jnp.maximum(cntc, 1.0) + cx * cx + cy * cy,
                        0.0)
        total = jnp.sum(per) * lam_ref[0, 0] * (0.5 / n)
        o_ref[...] = jnp.broadcast_to(total, (1, 1))

    return pl.pallas_call(
        body,
        out_shape=jax.ShapeDtypeStruct((1, 1), jnp.float32),
    )(partials, center_t, lam)


def kernel(feature, label, lambdas, center):
    n = feature.shape[0]
    partials = _sc_partials(feature, label, n=n)
    center_t = jnp.zeros((2, 16), jnp.float32).at[:, : center.shape[0]].set(
        center.T)
    lam = jnp.asarray(lambdas, jnp.float32).reshape(1, 1)
    loss = _tc_combine(partials, center_t, lam, n=n)
    return loss[0, 0]


# speedup vs baseline: 1.1852x; 1.1852x over previous
import functools

import jax
import jax.numpy as jnp
from jax import lax
from jax.experimental import pallas as pl
from jax.experimental.pallas import tpu as pltpu
from jax.experimental.pallas import tpu_sc as plsc


def _sc_partials(feature, label, *, n, num_workers=32):
    rows = n // 64
    base = rows // num_workers
    rem = rows % num_workers
    cg = 1
    for d in range(650, 0, -1):
        if base % d == 0:
            cg = d
            break
    k_chunks = base // cg

    mesh = plsc.VectorSubcoreMesh(
        core_axis_name="c", subcore_axis_name="s",
        num_cores=2, num_subcores=num_workers // 2)

    @functools.partial(
        pl.kernel,
        out_type=jax.ShapeDtypeStruct((num_workers, 4, 16), jnp.float32),
        mesh=mesh,
        compiler_params=pltpu.CompilerParams(needs_layout_passes=False),
        scratch_types=[
            pltpu.VMEM((cg * 64,), jnp.float32),
            pltpu.VMEM((cg * 64,), jnp.float32),
            pltpu.VMEM((cg * 64,), jnp.float32),
            pltpu.VMEM((32,), jnp.float32),
            pltpu.VMEM((16,), jnp.float32),
            pltpu.VMEM((16,), jnp.float32),
            pltpu.VMEM((32,), jnp.float32),
            pltpu.VMEM((16,), jnp.float32),
            pltpu.VMEM((16,), jnp.float32),
            pltpu.VMEM((4, 16), jnp.float32),
        ],
    )
    def sc_kernel(fx_hbm, fy_hbm, label_hbm, part_hbm, fxbuf, fybuf, lbuf,
                  s1, s2, cnt, s1b, s2b, cntb, obuf):
        wid = lax.axis_index("s") * 2 + lax.axis_index("c")
        rstart = wid * base + jnp.minimum(wid, rem)

        iota = lax.iota(jnp.int32, 16)
        dbl = iota + iota
        zeros = jnp.zeros((16,), jnp.float32)
        ones = jnp.ones((16,), jnp.float32)

        s1[pl.ds(0, 16)] = zeros
        s1[pl.ds(16, 16)] = zeros
        s2[...] = zeros
        cnt[...] = zeros
        s1b[pl.ds(0, 16)] = zeros
        s1b[pl.ds(16, 16)] = zeros
        s2b[...] = zeros
        cntb[...] = zeros

        def group_into(g, t1, t2, tc):
            b16 = g * 16
            lab = lbuf[pl.ds(b16, 16)]
            labi = lab.astype(jnp.int32)
            vx = fxbuf[pl.ds(b16, 16)]
            vy = fybuf[pl.ds(b16, 16)]
            b2 = labi + labi
            plsc.addupdate_scatter(tc, [labi], ones)
            plsc.addupdate_scatter(t1, [b2], vx)
            plsc.addupdate_scatter(t1, [b2 + 1], vy)
            plsc.addupdate_scatter(t2, [labi], vx * vx + vy * vy)

        def group_body(g, _):
            group_into(g, s1, s2, cnt)
            return _

        def pair_body(p, _):
            g0 = p + p
            group_into(g0, s1, s2, cnt)
            group_into(g0 + 1, s1b, s2b, cntb)
            return _

        def chunk_body(k, _):
            soff = (rstart + k * cg) * 64
            pltpu.sync_copy(fx_hbm.at[pl.ds(soff, cg * 64)], fxbuf)
            pltpu.sync_copy(fy_hbm.at[pl.ds(soff, cg * 64)], fybuf)
            pltpu.sync_copy(label_hbm.at[pl.ds(soff, cg * 64)], lbuf)

            @plsc.parallel_loop(0, cg * 2, step=1, unroll=2)
            def _groups(p):
                pair_body(p, None)
            return _

        lax.fori_loop(0, k_chunks, chunk_body, None)

        @pl.when(wid < rem)
        def _tail():
            soff = (rstart + base) * 64
            pltpu.sync_copy(fx_hbm.at[pl.ds(soff, 64)],
                            fxbuf.at[pl.ds(0, 64)])
            pltpu.sync_copy(fy_hbm.at[pl.ds(soff, 64)],
                            fybuf.at[pl.ds(0, 64)])
            pltpu.sync_copy(label_hbm.at[pl.ds(soff, 64)],
                            lbuf.at[pl.ds(0, 64)])
            lax.fori_loop(0, 4, group_body, None, unroll=2)

        s1[pl.ds(0, 16)] += s1b[pl.ds(0, 16)]
        s1[pl.ds(16, 16)] += s1b[pl.ds(16, 16)]
        s2[...] += s2b[...]
        cnt[...] += cntb[...]
        obuf[0, :] = plsc.load_gather(s1, [dbl])
        obuf[1, :] = plsc.load_gather(s1, [dbl + 1])
        obuf[2, :] = s2[...]
        obuf[3, :] = cnt[...]
        pltpu.sync_copy(obuf, part_hbm.at[wid])

    return sc_kernel(feature[:, 0], feature[:, 1], label)


def _tc_combine(partials, center_t, lam, *, n):
    def body(p_ref, ct_ref, lam_ref, o_ref):
        r = jnp.sum(p_ref[...], axis=0)
        s1x = r[0:1, :]
        s1y = r[1:2, :]
        s2c = r[2:3, :]
        cntc = r[3:4, :]
        cx = ct_ref[0:1, :]
        cy = ct_ref[1:2, :]
        num = s2c - 2.0 * (cx * s1x + cy * s1y)
        per = jnp.where(cntc > 0.0,
                        num / jnp.maximum(cntc, 1.0) + cx * cx + cy * cy,
                        0.0)
        total = jnp.sum(per) * lam_ref[0, 0] * (0.5 / n)
        o_ref[...] = jnp.broadcast_to(total, (1, 1))

    return pl.pallas_call(
        body,
        out_shape=jax.ShapeDtypeStruct((1, 1), jnp.float32),
    )(partials, center_t, lam)


def kernel(feature, label, lambdas, center):
    n = feature.shape[0]
    partials = _sc_partials(feature, label, n=n)
    center_t = jnp.zeros((2, 16), jnp.float32).at[:, : center.shape[0]].set(
        center.T)
    lam = jnp.asarray(lambdas, jnp.float32).reshape(1, 1)
    loss = _tc_combine(partials, center_t, lam, n=n)
    return loss[0, 0]


# speedup vs baseline: 1.1861x; 1.0007x over previous
import functools

import jax
import jax.numpy as jnp
from jax import lax
from jax.experimental import pallas as pl
from jax.experimental.pallas import tpu as pltpu
from jax.experimental.pallas import tpu_sc as plsc


def _sc_partials(feature, label, *, n, num_workers=32):
    rows = n // 64
    base = rows // num_workers
    rem = rows % num_workers
    cg = 1
    for d in range(650, 0, -1):
        if base % d == 0:
            cg = d
            break
    k_chunks = base // cg

    mesh = plsc.VectorSubcoreMesh(
        core_axis_name="c", subcore_axis_name="s",
        num_cores=2, num_subcores=num_workers // 2)

    @functools.partial(
        pl.kernel,
        out_type=jax.ShapeDtypeStruct((num_workers, 4, 16), jnp.float32),
        mesh=mesh,
        compiler_params=pltpu.CompilerParams(needs_layout_passes=False),
        scratch_types=[
            pltpu.VMEM((cg * 64,), jnp.float32),
            pltpu.VMEM((cg * 64,), jnp.float32),
            pltpu.VMEM((cg * 64,), jnp.float32),
            pltpu.VMEM((32,), jnp.float32),
            pltpu.VMEM((16,), jnp.float32),
            pltpu.VMEM((16,), jnp.float32),
            pltpu.VMEM((32,), jnp.float32),
            pltpu.VMEM((16,), jnp.float32),
            pltpu.VMEM((16,), jnp.float32),
            pltpu.VMEM((4, 16), jnp.float32),
        ],
    )
    def sc_kernel(fx_hbm, fy_hbm, label_hbm, part_hbm, fxbuf, fybuf, lbuf,
                  s1, s2, cnt, s1b, s2b, cntb, obuf):
        wid = lax.axis_index("s") * 2 + lax.axis_index("c")
        rstart = wid * base + jnp.minimum(wid, rem)

        iota = lax.iota(jnp.int32, 16)
        dbl = iota + iota
        zeros = jnp.zeros((16,), jnp.float32)
        ones = jnp.ones((16,), jnp.float32)

        s1[pl.ds(0, 16)] = zeros
        s1[pl.ds(16, 16)] = zeros
        s2[...] = zeros
        cnt[...] = zeros
        s1b[pl.ds(0, 16)] = zeros
        s1b[pl.ds(16, 16)] = zeros
        s2b[...] = zeros
        cntb[...] = zeros

        def group_into(g, t1, t2, tc):
            b16 = g * 16
            lab = lbuf[pl.ds(b16, 16)]
            labi = lab.astype(jnp.int32)
            vx = fxbuf[pl.ds(b16, 16)]
            vy = fybuf[pl.ds(b16, 16)]
            b2 = labi + labi
            plsc.addupdate_scatter(tc, [labi], ones)
            plsc.addupdate_scatter(t1, [b2], vx)
            plsc.addupdate_scatter(t1, [b2 + 1], vy)
            plsc.addupdate_scatter(t2, [labi], vx * vx + vy * vy)

        def group_body(g, _):
            group_into(g, s1, s2, cnt)
            return _

        def pair_body(p, _):
            g0 = p + p
            group_into(g0, s1, s2, cnt)
            group_into(g0 + 1, s1b, s2b, cntb)
            return _

        def chunk_body(k, _):
            soff = (rstart + k * cg) * 64
            pltpu.sync_copy(fx_hbm.at[pl.ds(soff, cg * 64)], fxbuf)
            pltpu.sync_copy(fy_hbm.at[pl.ds(soff, cg * 64)], fybuf)
            pltpu.sync_copy(label_hbm.at[pl.ds(soff, cg * 64)], lbuf)

            @plsc.parallel_loop(0, cg * 2, step=1, unroll=4)
            def _groups(p):
                pair_body(p, None)
            return _

        lax.fori_loop(0, k_chunks, chunk_body, None)

        @pl.when(wid < rem)
        def _tail():
            soff = (rstart + base) * 64
            pltpu.sync_copy(fx_hbm.at[pl.ds(soff, 64)],
                            fxbuf.at[pl.ds(0, 64)])
            pltpu.sync_copy(fy_hbm.at[pl.ds(soff, 64)],
                            fybuf.at[pl.ds(0, 64)])
            pltpu.sync_copy(label_hbm.at[pl.ds(soff, 64)],
                            lbuf.at[pl.ds(0, 64)])
            lax.fori_loop(0, 4, group_body, None, unroll=2)

        s1[pl.ds(0, 16)] += s1b[pl.ds(0, 16)]
        s1[pl.ds(16, 16)] += s1b[pl.ds(16, 16)]
        s2[...] += s2b[...]
        cnt[...] += cntb[...]
        obuf[0, :] = plsc.load_gather(s1, [dbl])
        obuf[1, :] = plsc.load_gather(s1, [dbl + 1])
        obuf[2, :] = s2[...]
        obuf[3, :] = cnt[...]
        pltpu.sync_copy(obuf, part_hbm.at[wid])

    return sc_kernel(feature[:, 0], feature[:, 1], label)


def _tc_combine(partials, center_t, lam, *, n):
    def body(p_ref, ct_ref, lam_ref, o_ref):
        r = jnp.sum(p_ref[...], axis=0)
        s1x = r[0:1, :]
        s1y = r[1:2, :]
        s2c = r[2:3, :]
        cntc = r[3:4, :]
        cx = ct_ref[0:1, :]
        cy = ct_ref[1:2, :]
        num = s2c - 2.0 * (cx * s1x + cy * s1y)
        per = jnp.where(cntc > 0.0,
                        num / jnp.maximum(cntc, 1.0) + cx * cx + cy * cy,
                        0.0)
        total = jnp.sum(per) * lam_ref[0, 0] * (0.5 / n)
        o_ref[...] = jnp.broadcast_to(total, (1, 1))

    return pl.pallas_call(
        body,
        out_shape=jax.ShapeDtypeStruct((1, 1), jnp.float32),
    )(partials, center_t, lam)


def kernel(feature, label, lambdas, center):
    n = feature.shape[0]
    partials = _sc_partials(feature, label, n=n)
    center_t = jnp.zeros((2, 16), jnp.float32).at[:, : center.shape[0]].set(
        center.T)
    lam = jnp.asarray(lambdas, jnp.float32).reshape(1, 1)
    loss = _tc_combine(partials, center_t, lam, n=n)
    return loss[0, 0]


# speedup vs baseline: 1.1868x; 1.0006x over previous
import functools

import jax
import jax.numpy as jnp
from jax import lax
from jax.experimental import pallas as pl
from jax.experimental.pallas import tpu as pltpu
from jax.experimental.pallas import tpu_sc as plsc


def _sc_partials(feature, label, *, n, num_workers=32):
    rows = n // 64
    base = rows // num_workers
    rem = rows % num_workers
    cg = 1
    for d in range(650, 0, -1):
        if base % d == 0:
            cg = d
            break
    k_chunks = base // cg

    mesh = plsc.VectorSubcoreMesh(
        core_axis_name="c", subcore_axis_name="s",
        num_cores=2, num_subcores=num_workers // 2)

    @functools.partial(
        pl.kernel,
        out_type=jax.ShapeDtypeStruct((num_workers, 4, 16), jnp.float32),
        mesh=mesh,
        compiler_params=pltpu.CompilerParams(needs_layout_passes=False),
        scratch_types=[
            pltpu.VMEM((cg * 64,), jnp.float32),
            pltpu.VMEM((cg * 64,), jnp.float32),
            pltpu.VMEM((cg * 64,), jnp.float32),
            pltpu.VMEM((32,), jnp.float32),
            pltpu.VMEM((16,), jnp.float32),
            pltpu.VMEM((16,), jnp.float32),
            pltpu.VMEM((32,), jnp.float32),
            pltpu.VMEM((16,), jnp.float32),
            pltpu.VMEM((16,), jnp.float32),
            pltpu.VMEM((4, 16), jnp.float32),
        ],
    )
    def sc_kernel(fx_hbm, fy_hbm, label_hbm, part_hbm, fxbuf, fybuf, lbuf,
                  s1, s2, cnt, s1b, s2b, cntb, obuf):
        wid = lax.axis_index("s") * 2 + lax.axis_index("c")
        rstart = wid * base + jnp.minimum(wid, rem)

        iota = lax.iota(jnp.int32, 16)
        dbl = iota + iota
        zeros = jnp.zeros((16,), jnp.float32)
        ones = jnp.ones((16,), jnp.float32)

        s1[pl.ds(0, 16)] = zeros
        s1[pl.ds(16, 16)] = zeros
        s2[...] = zeros
        cnt[...] = zeros
        s1b[pl.ds(0, 16)] = zeros
        s1b[pl.ds(16, 16)] = zeros
        s2b[...] = zeros
        cntb[...] = zeros

        def group_into(g, t1, t2, tc):
            b16 = g * 16
            lab = lbuf[pl.ds(b16, 16)]
            labi = lab.astype(jnp.int32)
            vx = fxbuf[pl.ds(b16, 16)]
            vy = fybuf[pl.ds(b16, 16)]
            b2 = labi + labi
            plsc.addupdate_scatter(tc, [labi], ones)
            plsc.addupdate_scatter(t1, [b2], vx)
            plsc.addupdate_scatter(t1, [b2 + 1], vy)
            plsc.addupdate_scatter(t2, [labi], vx * vx + vy * vy)

        def group_body(g, _):
            group_into(g, s1, s2, cnt)
            return _

        def pair_body(p, _):
            g0 = p + p
            group_into(g0, s1, s2, cnt)
            group_into(g0 + 1, s1b, s2b, cntb)
            return _

        def chunk_body(k, _):
            soff = (rstart + k * cg) * 64
            pltpu.sync_copy(fx_hbm.at[pl.ds(soff, cg * 64)], fxbuf)
            pltpu.sync_copy(fy_hbm.at[pl.ds(soff, cg * 64)], fybuf)
            pltpu.sync_copy(label_hbm.at[pl.ds(soff, cg * 64)], lbuf)

            @plsc.parallel_loop(0, cg * 2, step=1, unroll=4)
            def _groups(p):
                pair_body(p, None)
            return _

        lax.fori_loop(0, k_chunks, chunk_body, None)

        @pl.when(wid < rem)
        def _tail():
            soff = (rstart + base) * 64
            pltpu.sync_copy(fx_hbm.at[pl.ds(soff, 64)],
                            fxbuf.at[pl.ds(0, 64)])
            pltpu.sync_copy(fy_hbm.at[pl.ds(soff, 64)],
                            fybuf.at[pl.ds(0, 64)])
            pltpu.sync_copy(label_hbm.at[pl.ds(soff, 64)],
                            lbuf.at[pl.ds(0, 64)])
            lax.fori_loop(0, 4, group_body, None, unroll=2)

        s1[pl.ds(0, 16)] += s1b[pl.ds(0, 16)]
        s1[pl.ds(16, 16)] += s1b[pl.ds(16, 16)]
        s2[...] += s2b[...]
        cnt[...] += cntb[...]
        obuf[0, :] = plsc.load_gather(s1, [dbl])
        obuf[1, :] = plsc.load_gather(s1, [dbl + 1])
        obuf[2, :] = s2[...]
        obuf[3, :] = cnt[...]
        pltpu.sync_copy(obuf, part_hbm.at[wid])

    ft = feature.T
    return sc_kernel(ft[0], ft[1], label)


def _tc_combine(partials, center_t, lam, *, n):
    def body(p_ref, ct_ref, lam_ref, o_ref):
        r = jnp.sum(p_ref[...], axis=0)
        s1x = r[0:1, :]
        s1y = r[1:2, :]
        s2c = r[2:3, :]
        cntc = r[3:4, :]
        cx = ct_ref[0:1, :]
        cy = ct_ref[1:2, :]
        num = s2c - 2.0 * (cx * s1x + cy * s1y)
        per = jnp.where(cntc > 0.0,
                        num / jnp.maximum(cntc, 1.0) + cx * cx + cy * cy,
                        0.0)
        total = jnp.sum(per) * lam_ref[0, 0] * (0.5 / n)
        o_ref[...] = jnp.broadcast_to(total, (1, 1))

    return pl.pallas_call(
        body,
        out_shape=jax.ShapeDtypeStruct((1, 1), jnp.float32),
    )(partials, center_t, lam)


def kernel(feature, label, lambdas, center):
    n = feature.shape[0]
    partials = _sc_partials(feature, label, n=n)
    center_t = jnp.zeros((2, 16), jnp.float32).at[:, : center.shape[0]].set(
        center.T)
    lam = jnp.asarray(lambdas, jnp.float32).reshape(1, 1)
    loss = _tc_combine(partials, center_t, lam, n=n)
    return loss[0, 0]


# speedup vs baseline: 1.2505x; 1.0536x over previous
import functools

import jax
import jax.numpy as jnp
from jax import lax
from jax.experimental import pallas as pl
from jax.experimental.pallas import tpu as pltpu
from jax.experimental.pallas import tpu_sc as plsc


def _sc_partials(feature, label, *, n, num_workers=32):
    rows = n // 64
    base = rows // num_workers
    rem = rows % num_workers
    cg = 1
    for d in range(650, 0, -1):
        if base % d == 0:
            cg = d
            break
    k_chunks = base // cg

    mesh = plsc.VectorSubcoreMesh(
        core_axis_name="c", subcore_axis_name="s",
        num_cores=2, num_subcores=num_workers // 2)

    @functools.partial(
        pl.kernel,
        out_type=jax.ShapeDtypeStruct((num_workers, 4, 16), jnp.float32),
        mesh=mesh,
        compiler_params=pltpu.CompilerParams(needs_layout_passes=False),
        scratch_types=[
            pltpu.VMEM((cg * 64,), jnp.float32),
            pltpu.VMEM((cg * 64,), jnp.float32),
            pltpu.VMEM((cg * 64,), jnp.float32),
            pltpu.VMEM((64,), jnp.float32),
            pltpu.VMEM((32,), jnp.float32),
            pltpu.VMEM((32,), jnp.float32),
            pltpu.VMEM((64,), jnp.float32),
            pltpu.VMEM((32,), jnp.float32),
            pltpu.VMEM((32,), jnp.float32),
            pltpu.VMEM((4, 16), jnp.float32),
        ],
    )
    def sc_kernel(fx_hbm, fy_hbm, label_hbm, part_hbm, fxbuf, fybuf, lbuf,
                  s1, s2, cnt, s1b, s2b, cntb, obuf):
        wid = lax.axis_index("s") * 2 + lax.axis_index("c")
        rstart = wid * base + jnp.minimum(wid, rem)

        iota = lax.iota(jnp.int32, 16)
        dbl = iota + iota
        par = lax.bitwise_and(iota, 1)
        zeros = jnp.zeros((16,), jnp.float32)
        ones = jnp.ones((16,), jnp.float32)

        for t in (s1, s1b):
            for o in (0, 16, 32, 48):
                t[pl.ds(o, 16)] = zeros
        for t in (s2, cnt, s2b, cntb):
            t[pl.ds(0, 16)] = zeros
            t[pl.ds(16, 16)] = zeros

        def group_into(g, t1, t2, tc):
            b16 = g * 16
            lab = lbuf[pl.ds(b16, 16)]
            labi = lab.astype(jnp.int32)
            vx = fxbuf[pl.ds(b16, 16)]
            vy = fybuf[pl.ds(b16, 16)]
            b2 = labi + labi + par
            b4 = b2 + b2
            plsc.addupdate_scatter(tc, [b2], ones)
            plsc.addupdate_scatter(t1, [b4], vx)
            plsc.addupdate_scatter(t1, [b4 + 1], vy)
            plsc.addupdate_scatter(t2, [b2], vx * vx + vy * vy)

        def group_body(g, _):
            group_into(g, s1, s2, cnt)
            return _

        def pair_body(p, _):
            g0 = p + p
            group_into(g0, s1, s2, cnt)
            group_into(g0 + 1, s1b, s2b, cntb)
            return _

        def chunk_body(k, _):
            soff = (rstart + k * cg) * 64
            pltpu.sync_copy(fx_hbm.at[pl.ds(soff, cg * 64)], fxbuf)
            pltpu.sync_copy(fy_hbm.at[pl.ds(soff, cg * 64)], fybuf)
            pltpu.sync_copy(label_hbm.at[pl.ds(soff, cg * 64)], lbuf)

            @plsc.parallel_loop(0, cg * 2, step=1, unroll=4)
            def _groups(p):
                pair_body(p, None)
            return _

        lax.fori_loop(0, k_chunks, chunk_body, None)

        @pl.when(wid < rem)
        def _tail():
            soff = (rstart + base) * 64
            pltpu.sync_copy(fx_hbm.at[pl.ds(soff, 64)],
                            fxbuf.at[pl.ds(0, 64)])
            pltpu.sync_copy(fy_hbm.at[pl.ds(soff, 64)],
                            fybuf.at[pl.ds(0, 64)])
            pltpu.sync_copy(label_hbm.at[pl.ds(soff, 64)],
                            lbuf.at[pl.ds(0, 64)])
            lax.fori_loop(0, 4, group_body, None, unroll=2)

        for o in (0, 16, 32, 48):
            s1[pl.ds(o, 16)] += s1b[pl.ds(o, 16)]
        for o in (0, 16):
            s2[pl.ds(o, 16)] += s2b[pl.ds(o, 16)]
            cnt[pl.ds(o, 16)] += cntb[pl.ds(o, 16)]
        q = dbl + dbl
        obuf[0, :] = (plsc.load_gather(s1, [q]) +
                      plsc.load_gather(s1, [q + 1]))
        obuf[1, :] = (plsc.load_gather(s1, [q + 2]) +
                      plsc.load_gather(s1, [q + 3]))
        obuf[2, :] = (plsc.load_gather(s2, [dbl]) +
                      plsc.load_gather(s2, [dbl + 1]))
        obuf[3, :] = (plsc.load_gather(cnt, [dbl]) +
                      plsc.load_gather(cnt, [dbl + 1]))
        pltpu.sync_copy(obuf, part_hbm.at[wid])

    return sc_kernel(feature[:, 0], feature[:, 1], label)


def _tc_combine(partials, center_t, lam, *, n):
    def body(p_ref, ct_ref, lam_ref, o_ref):
        r = jnp.sum(p_ref[...], axis=0)
        s1x = r[0:1, :]
        s1y = r[1:2, :]
        s2c = r[2:3, :]
        cntc = r[3:4, :]
        cx = ct_ref[0:1, :]
        cy = ct_ref[1:2, :]
        num = s2c - 2.0 * (cx * s1x + cy * s1y)
        per = jnp.where(cntc > 0.0,
                        num / jnp.maximum(cntc, 1.0) + cx * cx + cy * cy,
                        0.0)
        total = jnp.sum(per) * lam_ref[0, 0] * (0.5 / n)
        o_ref[...] = jnp.broadcast_to(total, (1, 1))

    return pl.pallas_call(
        body,
        out_shape=jax.ShapeDtypeStruct((1, 1), jnp.float32),
    )(partials, center_t, lam)


def kernel(feature, label, lambdas, center):
    n = feature.shape[0]
    partials = _sc_partials(feature, label, n=n)
    center_t = jnp.zeros((2, 16), jnp.float32).at[:, : center.shape[0]].set(
        center.T)
    lam = jnp.asarray(lambdas, jnp.float32).reshape(1, 1)
    loss = _tc_combine(partials, center_t, lam, n=n)
    return loss[0, 0]


# speedup vs baseline: 1.3104x; 1.0479x over previous
import functools

import jax
import jax.numpy as jnp
from jax import lax
from jax.experimental import pallas as pl
from jax.experimental.pallas import tpu as pltpu
from jax.experimental.pallas import tpu_sc as plsc


def _sc_partials(feature, label, *, n, num_workers=32):
    rows = n // 64
    base = rows // num_workers
    rem = rows % num_workers
    cg = 1
    for d in range(330, 0, -1):
        if base % d == 0:
            cg = d
            break
    k_chunks = base // cg

    mesh = plsc.VectorSubcoreMesh(
        core_axis_name="c", subcore_axis_name="s",
        num_cores=2, num_subcores=num_workers // 2)

    @functools.partial(
        pl.kernel,
        out_type=jax.ShapeDtypeStruct((num_workers, 4, 16), jnp.float32),
        mesh=mesh,
        compiler_params=pltpu.CompilerParams(needs_layout_passes=False),
        scratch_types=[
            pltpu.VMEM((cg * 64,), jnp.float32),
            pltpu.VMEM((cg * 64,), jnp.float32),
            pltpu.VMEM((cg * 64,), jnp.float32),
            pltpu.VMEM((cg * 64,), jnp.float32),
            pltpu.VMEM((cg * 64,), jnp.float32),
            pltpu.VMEM((cg * 64,), jnp.float32),
            pltpu.SemaphoreType.DMA,
            pltpu.SemaphoreType.DMA,
            pltpu.VMEM((64,), jnp.float32),
            pltpu.VMEM((32,), jnp.float32),
            pltpu.VMEM((32,), jnp.float32),
            pltpu.VMEM((64,), jnp.float32),
            pltpu.VMEM((32,), jnp.float32),
            pltpu.VMEM((32,), jnp.float32),
            pltpu.VMEM((4, 16), jnp.float32),
        ],
    )
    def sc_kernel(fx_hbm, fy_hbm, label_hbm, part_hbm, fxbuf, fybuf, lbuf,
                  fxbuf2, fybuf2, lbuf2, sema, semb,
                  s1, s2, cnt, s1b, s2b, cntb, obuf):
        wid = lax.axis_index("s") * 2 + lax.axis_index("c")
        rstart = wid * base + jnp.minimum(wid, rem)

        iota = lax.iota(jnp.int32, 16)
        dbl = iota + iota
        par = lax.bitwise_and(iota, 1)
        zeros = jnp.zeros((16,), jnp.float32)
        ones = jnp.ones((16,), jnp.float32)

        for t in (s1, s1b):
            for o in (0, 16, 32, 48):
                t[pl.ds(o, 16)] = zeros
        for t in (s2, cnt, s2b, cntb):
            t[pl.ds(0, 16)] = zeros
            t[pl.ds(16, 16)] = zeros

        def group_into(g, t1, t2, tc, fx, fy, lb):
            b16 = g * 16
            lab = lb[pl.ds(b16, 16)]
            labi = lab.astype(jnp.int32)
            vx = fx[pl.ds(b16, 16)]
            vy = fy[pl.ds(b16, 16)]
            b2 = labi + labi + par
            b4 = b2 + b2
            plsc.addupdate_scatter(tc, [b2], ones)
            plsc.addupdate_scatter(t1, [b4], vx)
            plsc.addupdate_scatter(t1, [b4 + 1], vy)
            plsc.addupdate_scatter(t2, [b2], vx * vx + vy * vy)

        sets = ((fxbuf, fybuf, lbuf, sema), (fxbuf2, fybuf2, lbuf2, semb))

        def issue(k):
            fx, fy, lb, sem = sets[k % 2]
            soff = (rstart + k * cg) * 64
            return (
                pltpu.async_copy(fx_hbm.at[pl.ds(soff, cg * 64)], fx, sem),
                pltpu.async_copy(fy_hbm.at[pl.ds(soff, cg * 64)], fy, sem),
                pltpu.async_copy(label_hbm.at[pl.ds(soff, cg * 64)], lb, sem),
            )

        handles = issue(0)
        for k in range(k_chunks):
            nxt = issue(k + 1) if k + 1 < k_chunks else None
            for h in handles:
                h.wait()
            fx, fy, lb, _sem = sets[k % 2]

            @plsc.parallel_loop(0, cg * 2, step=1, unroll=4)
            def _groups(p):
                g0 = p + p
                group_into(g0, s1, s2, cnt, fx, fy, lb)
                group_into(g0 + 1, s1b, s2b, cntb, fx, fy, lb)
            handles = nxt

        @pl.when(wid < rem)
        def _tail():
            soff = (rstart + base) * 64
            pltpu.sync_copy(fx_hbm.at[pl.ds(soff, 64)],
                            fxbuf.at[pl.ds(0, 64)])
            pltpu.sync_copy(fy_hbm.at[pl.ds(soff, 64)],
                            fybuf.at[pl.ds(0, 64)])
            pltpu.sync_copy(label_hbm.at[pl.ds(soff, 64)],
                            lbuf.at[pl.ds(0, 64)])

            def tail_body(g, _):
                group_into(g, s1, s2, cnt, fxbuf, fybuf, lbuf)
                return _
            lax.fori_loop(0, 4, tail_body, None, unroll=2)

        for o in (0, 16, 32, 48):
            s1[pl.ds(o, 16)] += s1b[pl.ds(o, 16)]
        for o in (0, 16):
            s2[pl.ds(o, 16)] += s2b[pl.ds(o, 16)]
            cnt[pl.ds(o, 16)] += cntb[pl.ds(o, 16)]
        q = dbl + dbl
        obuf[0, :] = (plsc.load_gather(s1, [q]) +
                      plsc.load_gather(s1, [q + 1]))
        obuf[1, :] = (plsc.load_gather(s1, [q + 2]) +
                      plsc.load_gather(s1, [q + 3]))
        obuf[2, :] = (plsc.load_gather(s2, [dbl]) +
                      plsc.load_gather(s2, [dbl + 1]))
        obuf[3, :] = (plsc.load_gather(cnt, [dbl]) +
                      plsc.load_gather(cnt, [dbl + 1]))
        pltpu.sync_copy(obuf, part_hbm.at[wid])

    return sc_kernel(feature[:, 0], feature[:, 1], label)


def _tc_combine(partials, center_t, lam, *, n):
    def body(p_ref, ct_ref, lam_ref, o_ref):
        r = jnp.sum(p_ref[...], axis=0)
        s1x = r[0:1, :]
        s1y = r[1:2, :]
        s2c = r[2:3, :]
        cntc = r[3:4, :]
        cx = ct_ref[0:1, :]
        cy = ct_ref[1:2, :]
        num = s2c - 2.0 * (cx * s1x + cy * s1y)
        per = jnp.where(cntc > 0.0,
                        num / jnp.maximum(cntc, 1.0) + cx * cx + cy * cy,
                        0.0)
        total = jnp.sum(per) * lam_ref[0, 0] * (0.5 / n)
        o_ref[...] = jnp.broadcast_to(total, (1, 1))

    return pl.pallas_call(
        body,
        out_shape=jax.ShapeDtypeStruct((1, 1), jnp.float32),
    )(partials, center_t, lam)


def kernel(feature, label, lambdas, center):
    n = feature.shape[0]
    partials = _sc_partials(feature, label, n=n)
    center_t = jnp.zeros((2, 16), jnp.float32).at[:, : center.shape[0]].set(
        center.T)
    lam = jnp.asarray(lambdas, jnp.float32).reshape(1, 1)
    loss = _tc_combine(partials, center_t, lam, n=n)
    return loss[0, 0]


# speedup vs baseline: 2.0702x; 1.5799x over previous
import functools

import jax
import jax.numpy as jnp
from jax import lax
from jax.experimental import pallas as pl
from jax.experimental.pallas import tpu as pltpu
from jax.experimental.pallas import tpu_sc as plsc


def _sc_partials(feature, label, *, n, num_workers=32):
    rows = n // 64
    base = rows // num_workers
    rem = rows % num_workers
    cg = 1
    for d in range(330, 0, -1):
        if base % d == 0:
            cg = d
            break
    k_chunks = base // cg

    mesh = plsc.VectorSubcoreMesh(
        core_axis_name="c", subcore_axis_name="s",
        num_cores=2, num_subcores=num_workers // 2)

    @functools.partial(
        pl.kernel,
        out_type=jax.ShapeDtypeStruct((num_workers, 4, 16), jnp.float32),
        mesh=mesh,
        compiler_params=pltpu.CompilerParams(needs_layout_passes=False),
        scratch_types=[
            pltpu.VMEM((cg * 64,), jnp.float32),
            pltpu.VMEM((cg * 64,), jnp.float32),
            pltpu.VMEM((cg * 64,), jnp.float32),
            pltpu.VMEM((cg * 64,), jnp.float32),
            pltpu.VMEM((cg * 64,), jnp.float32),
            pltpu.VMEM((cg * 64,), jnp.float32),
            pltpu.SemaphoreType.DMA,
            pltpu.SemaphoreType.DMA,
            pltpu.VMEM((64,), jnp.float32),
            pltpu.VMEM((32,), jnp.float32),
            pltpu.VMEM((32,), jnp.float32),
            pltpu.VMEM((64,), jnp.float32),
            pltpu.VMEM((32,), jnp.float32),
            pltpu.VMEM((32,), jnp.float32),
            pltpu.VMEM((4, 16), jnp.float32),
        ],
    )
    def sc_kernel(fx_hbm, fy_hbm, label_hbm, part_hbm, fxbuf, fybuf, lbuf,
                  fxbuf2, fybuf2, lbuf2, sema, semb,
                  s1, s2, cnt, s1b, s2b, cntb, obuf):
        wid = lax.axis_index("s") * 2 + lax.axis_index("c")
        rstart = wid * base + jnp.minimum(wid, rem)

        iota = lax.iota(jnp.int32, 16)
        dbl = iota + iota
        par = lax.bitwise_and(iota, 1)
        zeros = jnp.zeros((16,), jnp.float32)
        ones = jnp.ones((16,), jnp.float32)

        for t in (s1, s1b):
            for o in (0, 16, 32, 48):
                t[pl.ds(o, 16)] = zeros
        for t in (s2, cnt, s2b, cntb):
            t[pl.ds(0, 16)] = zeros
            t[pl.ds(16, 16)] = zeros

        def group_into(g, t1, t2, tc, fx, fy, lb):
            b16 = g * 16
            lab = lb[pl.ds(b16, 16)]
            labi = lab.astype(jnp.int32)
            vx = fx[pl.ds(b16, 16)]
            vy = fy[pl.ds(b16, 16)]
            b2 = labi + labi + par
            b4 = b2 + b2
            plsc.addupdate_scatter(tc, [b2], ones)
            plsc.addupdate_scatter(t1, [b4], vx)
            plsc.addupdate_scatter(t1, [b4 + 1], vy)
            plsc.addupdate_scatter(t2, [b2], vx * vx + vy * vy)

        sets = ((fxbuf, fybuf, lbuf, sema), (fxbuf2, fybuf2, lbuf2, semb))

        def issue(k):
            fx, fy, lb, sem = sets[k % 2]
            soff = (rstart + k * cg) * 64
            return (
                pltpu.async_copy(fx_hbm.at[pl.ds(soff, cg * 64)], fx, sem),
                pltpu.async_copy(fy_hbm.at[pl.ds(n + soff, cg * 64)], fy, sem),
                pltpu.async_copy(label_hbm.at[pl.ds(soff, cg * 64)], lb, sem),
            )

        handles = issue(0)
        for k in range(k_chunks):
            nxt = issue(k + 1) if k + 1 < k_chunks else None
            for h in handles:
                h.wait()
            fx, fy, lb, _sem = sets[k % 2]

            @plsc.parallel_loop(0, cg * 2, step=1, unroll=4)
            def _groups(p):
                g0 = p + p
                group_into(g0, s1, s2, cnt, fx, fy, lb)
                group_into(g0 + 1, s1b, s2b, cntb, fx, fy, lb)
            handles = nxt

        @pl.when(wid < rem)
        def _tail():
            soff = (rstart + base) * 64
            pltpu.sync_copy(fx_hbm.at[pl.ds(soff, 64)],
                            fxbuf.at[pl.ds(0, 64)])
            pltpu.sync_copy(fy_hbm.at[pl.ds(n + soff, 64)],
                            fybuf.at[pl.ds(0, 64)])
            pltpu.sync_copy(label_hbm.at[pl.ds(soff, 64)],
                            lbuf.at[pl.ds(0, 64)])

            def tail_body(g, _):
                group_into(g, s1, s2, cnt, fxbuf, fybuf, lbuf)
                return _
            lax.fori_loop(0, 4, tail_body, None, unroll=2)

        for o in (0, 16, 32, 48):
            s1[pl.ds(o, 16)] += s1b[pl.ds(o, 16)]
        for o in (0, 16):
            s2[pl.ds(o, 16)] += s2b[pl.ds(o, 16)]
            cnt[pl.ds(o, 16)] += cntb[pl.ds(o, 16)]
        q = dbl + dbl
        obuf[0, :] = (plsc.load_gather(s1, [q]) +
                      plsc.load_gather(s1, [q + 1]))
        obuf[1, :] = (plsc.load_gather(s1, [q + 2]) +
                      plsc.load_gather(s1, [q + 3]))
        obuf[2, :] = (plsc.load_gather(s2, [dbl]) +
                      plsc.load_gather(s2, [dbl + 1]))
        obuf[3, :] = (plsc.load_gather(cnt, [dbl]) +
                      plsc.load_gather(cnt, [dbl + 1]))
        pltpu.sync_copy(obuf, part_hbm.at[wid])

    fxy = feature.T.reshape(-1)
    return sc_kernel(fxy, fxy, label)


def _tc_combine(partials, center_t, lam, *, n):
    def body(p_ref, ct_ref, lam_ref, o_ref):
        r = jnp.sum(p_ref[...], axis=0)
        s1x = r[0:1, :]
        s1y = r[1:2, :]
        s2c = r[2:3, :]
        cntc = r[3:4, :]
        cx = ct_ref[0:1, :]
        cy = ct_ref[1:2, :]
        num = s2c - 2.0 * (cx * s1x + cy * s1y)
        per = jnp.where(cntc > 0.0,
                        num / jnp.maximum(cntc, 1.0) + cx * cx + cy * cy,
                        0.0)
        total = jnp.sum(per) * lam_ref[0, 0] * (0.5 / n)
        o_ref[...] = jnp.broadcast_to(total, (1, 1))

    return pl.pallas_call(
        body,
        out_shape=jax.ShapeDtypeStruct((1, 1), jnp.float32),
    )(partials, center_t, lam)


def kernel(feature, label, lambdas, center):
    n = feature.shape[0]
    partials = _sc_partials(feature, label, n=n)
    center_t = jnp.zeros((2, 16), jnp.float32).at[:, : center.shape[0]].set(
        center.T)
    lam = jnp.asarray(lambdas, jnp.float32).reshape(1, 1)
    loss = _tc_combine(partials, center_t, lam, n=n)
    return loss[0, 0]
